# baseline (speedup 1.0000x reference)
import functools

import jax
import jax.numpy as jnp
from jax import lax
from jax.experimental import pallas as pl
from jax.experimental.pallas import tpu as pltpu
from jax.experimental.pallas import tpu_sc as plsc

NC = 2
NS = 16
LANES = 16
C = 128
F32 = jnp.float32


def _mesh():
    return plsc.VectorSubcoreMesh(
        core_axis_name="c", subcore_axis_name="s", num_cores=NC, num_subcores=NS
    )


def _sc_degrees(src2, dst2, np_rows):
    nch = src2.shape[0]
    npt = np_rows // NS

    @functools.partial(
        pl.kernel,
        out_type=jax.ShapeDtypeStruct((2 * NC, np_rows), F32),
        mesh=_mesh(),
        scratch_types=[
            pltpu.VMEM((C,), jnp.int32),
            pltpu.VMEM((C,), F32),
            pltpu.VMEM((npt,), F32),
            pltpu.VMEM_SHARED((np_rows,), F32),
            pltpu.VMEM_SHARED((np_rows,), F32),
        ],
    )
    def k(src_hbm, dst_hbm, out_hbm, idx_v, ones_v, stage_v, degs_sp, degd_sp):
        c = lax.axis_index("c")
        s = lax.axis_index("s")
        wid = s * NC + c
        for j in range(C // LANES):
            ones_v[pl.ds(j * LANES, LANES)] = jnp.full((LANES,), 1.0, F32)
        for j in range(npt // LANES):
            stage_v[pl.ds(j * LANES, LANES)] = jnp.zeros((LANES,), F32)
        pltpu.sync_copy(stage_v, degs_sp.at[pl.ds(s * npt, npt)])
        pltpu.sync_copy(stage_v, degd_sp.at[pl.ds(s * npt, npt)])
        plsc.subcore_barrier()

        @pl.loop(wid, nch, step=NC * NS)
        def _(ch):
            pltpu.sync_copy(src_hbm.at[ch], idx_v)
            pltpu.sync_copy(ones_v, degs_sp.at[idx_v], add=True)
            pltpu.sync_copy(dst_hbm.at[ch], idx_v)
            pltpu.sync_copy(ones_v, degd_sp.at[idx_v], add=True)

        plsc.subcore_barrier()
        pltpu.sync_copy(degs_sp.at[pl.ds(s * npt, npt)], stage_v)
        pltpu.sync_copy(stage_v, out_hbm.at[c, pl.ds(s * npt, npt)])
        pltpu.sync_copy(degd_sp.at[pl.ds(s * npt, npt)], stage_v)
        pltpu.sync_copy(stage_v, out_hbm.at[NC + c, pl.ds(s * npt, npt)])

    return k(src2, dst2)


AGG_G = 16


def _sc_aggregate(feat, src2, dst2):
    np_rows, d = feat.shape
    nch = src2.shape[0]
    rpt = np_rows // NS
    sb = 32
    nsb = rpt // sb
    nw = NC * NS
    nmy = nch // nw
    ngr = nmy // AGG_G

    @functools.partial(
        pl.kernel,
        out_type=jax.ShapeDtypeStruct((NC, np_rows, d), F32),
        mesh=_mesh(),
        scratch_types=[
            pltpu.VMEM((AGG_G, C), jnp.int32),
            pltpu.VMEM((AGG_G, C), jnp.int32),
            pltpu.VMEM((C, d), F32),
            pltpu.VMEM((C, d), F32),
            pltpu.VMEM((sb, d), F32),
            pltpu.VMEM_SHARED((np_rows, d), F32),
            pltpu.SemaphoreType.DMA,
            pltpu.SemaphoreType.DMA,
            pltpu.SemaphoreType.DMA,
        ],
    )
    def k(feat_hbm, src_hbm, dst_hbm, out_hbm, sidx_g, didx_g, rows_a, rows_b,
          stage_v, acc_sp, sem_i, sem_g, sem_s):
        c = lax.axis_index("c")
        s = lax.axis_index("s")
        wid = s * NC + c
        rows = (rows_a, rows_b)

        @pl.loop(0, sb)
        def _(i):
            for j in range(d // LANES):
                stage_v[i, pl.ds(j * LANES, LANES)] = jnp.zeros((LANES,), F32)

        @pl.loop(0, nsb)
        def _(t):
            pltpu.sync_copy(stage_v, acc_sp.at[pl.ds(s * rpt + t * sb, sb)])

        plsc.subcore_barrier()

        @pl.loop(0, ngr)
        def _(g):
            base = wid * nmy + g * AGG_G
            hi1 = pltpu.async_copy(src_hbm.at[pl.ds(base, AGG_G)], sidx_g, sem_i)
            hi2 = pltpu.async_copy(dst_hbm.at[pl.ds(base, AGG_G)], didx_g, sem_i)
            hi1.wait()
            hi2.wait()
            hs = [None, None]
            for j in range(AGG_G):
                b = j % 2
                if hs[b] is not None:
                    hs[b].wait()
                pltpu.async_copy(feat_hbm.at[sidx_g.at[j]], rows[b], sem_g).wait()
                hs[b] = pltpu.async_copy(rows[b], acc_sp.at[didx_g.at[j]],
                                         sem_s, add=True)
            hs[0].wait()
            hs[1].wait()

        plsc.subcore_barrier()

        @pl.loop(0, nsb)
        def _(t):
            pltpu.sync_copy(acc_sp.at[pl.ds(s * rpt + t * sb, sb)], stage_v)
            pltpu.sync_copy(stage_v, out_hbm.at[c, pl.ds(s * rpt + t * sb, sb)])

    return k(feat, src2, dst2)


def _sc_gather_rows(table, idx2):
    np_rows, d = table.shape
    nch = idx2.shape[0]
    rpt = np_rows // NS
    sb = 64
    nsb = rpt // sb

    @functools.partial(
        pl.kernel,
        out_type=jax.ShapeDtypeStruct((nch * C, d), F32),
        mesh=_mesh(),
        scratch_types=[
            pltpu.VMEM((C,), jnp.int32),
            pltpu.VMEM((C, d), F32),
            pltpu.VMEM_SHARED((np_rows, d), F32),
        ],
    )
    def k(tab_hbm, idx_hbm, out_hbm, idx_v, rows_v, tab_sp):
        c = lax.axis_index("c")
        s = lax.axis_index("s")
        wid = s * NC + c

        @pl.loop(0, nsb)
        def _(t):
            pltpu.sync_copy(tab_hbm.at[pl.ds(s * rpt + t * sb, sb)], rows_v.at[pl.ds(0, sb)])
            pltpu.sync_copy(rows_v.at[pl.ds(0, sb)], tab_sp.at[pl.ds(s * rpt + t * sb, sb)])

        plsc.subcore_barrier()

        @pl.loop(wid, nch, step=NC * NS)
        def _(ch):
            pltpu.sync_copy(idx_hbm.at[ch], idx_v)
            pltpu.sync_copy(tab_sp.at[idx_v], rows_v)
            pltpu.sync_copy(rows_v, out_hbm.at[pl.ds(ch * C, C)])

    return k(table, idx2)


def _tc_norms_feat(degparts, h_pad):
    np_rows, d = h_pad.shape

    def body(dp_ref, h_ref, feat_ref, norms_ref):
        deg_s = dp_ref[0, :] + dp_ref[1, :]
        deg_d = dp_ref[2, :] + dp_ref[3, :]
        ns = lax.rsqrt(jnp.maximum(deg_s, 1.0))
        nd = lax.rsqrt(jnp.maximum(deg_d, 1.0))
        feat_ref[...] = h_ref[...] * ns[:, None]
        norms_ref[0, :] = ns
        norms_ref[1, :] = nd

    return pl.pallas_call(
        body,
        out_shape=(
            jax.ShapeDtypeStruct((np_rows, d), F32),
            jax.ShapeDtypeStruct((2, np_rows), F32),
        ),
    )(degparts, h_pad)


def _tc_layer(parts, norms, w, b2d, scale_src):
    np_rows, d = parts.shape[1], parts.shape[2]

    def body(p_ref, n_ref, w_ref, b_ref, o_ref):
        agg = (p_ref[0] + p_ref[1]) * n_ref[1, :][:, None]
        out = jnp.dot(agg, w_ref[...], preferred_element_type=F32,
                      precision=lax.Precision.HIGHEST)
        out = jnp.maximum(out + b_ref[...], 0.0)
        if scale_src:
            out = out * n_ref[0, :][:, None]
        o_ref[...] = out

    return pl.pallas_call(
        body,
        out_shape=jax.ShapeDtypeStruct((np_rows, w.shape[1]), F32),
    )(parts, norms, w, b2d)


def _tc_pairs(g, w3, b3_2d, p_count):
    d = g.shape[1]
    ncls = w3.shape[1]
    bp = 2000
    nblk = p_count // bp
    off2 = p_count // bp

    def body(h1_ref, h2_ref, w_ref, b_ref, o_ref):
        h1 = h1_ref[...]
        h2 = h2_ref[...]
        w = w_ref[...]
        out = jnp.dot(h1, w[0:d], preferred_element_type=F32,
                      precision=lax.Precision.HIGHEST)
        out += jnp.dot(h2, w[d:2 * d], preferred_element_type=F32,
                       precision=lax.Precision.HIGHEST)
        out += jnp.dot(jnp.abs(h1 - h2), w[2 * d:3 * d], preferred_element_type=F32,
                       precision=lax.Precision.HIGHEST)
        o_ref[...] = out + b_ref[...]

    return pl.pallas_call(
        body,
        grid=(nblk,),
        in_specs=[
            pl.BlockSpec((bp, d), lambda i: (i, 0)),
            pl.BlockSpec((bp, d), lambda i: (i + off2, 0)),
            pl.BlockSpec((3 * d, ncls), lambda i: (0, 0)),
            pl.BlockSpec((1, ncls), lambda i: (0, 0)),
        ],
        out_specs=pl.BlockSpec((bp, ncls), lambda i: (i, 0)),
        out_shape=jax.ShapeDtypeStruct((p_count, ncls), F32),
    )(g, g, w3, b3_2d)


def _pad_chunk_idx(idx, pad_value, chunk_mult=1):
    n = idx.shape[0]
    unit = C * chunk_mult
    n_pad = -(-n // unit) * unit
    if n_pad != n:
        idx = jnp.concatenate(
            [idx, jnp.full((n_pad - n,), pad_value, jnp.int32)])
    return idx.reshape(-1, C)


def kernel(h, edge_index, x1, x2, W1, b1, W2, b2, W3, b3):
    n, d = h.shape
    p_count = x1.shape[0]

    np_rows = -(-n // (NS * LANES * 8)) * (NS * LANES * 8)
    h_pad = jnp.pad(h, ((0, np_rows - n), (0, 0)))

    src2 = _pad_chunk_idx(edge_index[0], n, NC * NS * AGG_G)
    dst2 = _pad_chunk_idx(edge_index[1], n, NC * NS * AGG_G)
    x12 = _pad_chunk_idx(jnp.concatenate([x1, x2]), 0)

    degparts = _sc_degrees(src2, dst2, np_rows)
    feat1, norms = _tc_norms_feat(degparts, h_pad)
    parts1 = _sc_aggregate(feat1, src2, dst2)
    feat2 = _tc_layer(parts1, norms, W1, b1.reshape(1, -1), True)
    parts2 = _sc_aggregate(feat2, src2, dst2)
    h2arr = _tc_layer(parts2, norms, W2, b2.reshape(1, -1), False)
    g = _sc_gather_rows(h2arr, x12)
    return _tc_pairs(g, W3, b3.reshape(1, -1), p_count)


# speedup vs baseline: 1.7618x; 1.7618x over previous
import functools

import jax
import jax.numpy as jnp
from jax import lax
from jax.experimental import pallas as pl
from jax.experimental.pallas import tpu as pltpu
from jax.experimental.pallas import tpu_sc as plsc

NC = 2
NS = 16
LANES = 16
C = 128
F32 = jnp.float32


def _mesh():
    return plsc.VectorSubcoreMesh(
        core_axis_name="c", subcore_axis_name="s", num_cores=NC, num_subcores=NS
    )


def _sc_degrees(sd2, np_rows):
    nch = sd2.shape[0]
    npt = np_rows // NS

    @functools.partial(
        pl.kernel,
        out_type=jax.ShapeDtypeStruct((2 * NC, np_rows), F32),
        mesh=_mesh(),
        scratch_types=[
            pltpu.VMEM((2, C), jnp.int32),
            pltpu.VMEM((C,), F32),
            pltpu.VMEM((npt,), F32),
            pltpu.VMEM_SHARED((np_rows,), F32),
            pltpu.VMEM_SHARED((np_rows,), F32),
        ],
    )
    def k(sd_hbm, out_hbm, idx_v, ones_v, stage_v, degs_sp, degd_sp):
        c = lax.axis_index("c")
        s = lax.axis_index("s")
        wid = s * NC + c
        for j in range(C // LANES):
            ones_v[pl.ds(j * LANES, LANES)] = jnp.full((LANES,), 1.0, F32)
        for j in range(npt // LANES):
            stage_v[pl.ds(j * LANES, LANES)] = jnp.zeros((LANES,), F32)
        pltpu.sync_copy(stage_v, degs_sp.at[pl.ds(s * npt, npt)])
        pltpu.sync_copy(stage_v, degd_sp.at[pl.ds(s * npt, npt)])
        plsc.subcore_barrier()

        @pl.loop(wid, nch, step=NC * NS)
        def _(ch):
            pltpu.sync_copy(sd_hbm.at[ch], idx_v)
            pltpu.sync_copy(ones_v, degs_sp.at[idx_v.at[0]], add=True)
            pltpu.sync_copy(ones_v, degd_sp.at[idx_v.at[1]], add=True)

        plsc.subcore_barrier()
        pltpu.sync_copy(degs_sp.at[pl.ds(s * npt, npt)], stage_v)
        pltpu.sync_copy(stage_v, out_hbm.at[c, pl.ds(s * npt, npt)])
        pltpu.sync_copy(degd_sp.at[pl.ds(s * npt, npt)], stage_v)
        pltpu.sync_copy(stage_v, out_hbm.at[NC + c, pl.ds(s * npt, npt)])

    return k(sd2)


def _sc_aggregate(feat, src2, dst2):
    np_rows, d = feat.shape
    nch = src2.shape[0]
    rpt = np_rows // NS
    sb = 64
    nsb = rpt // sb

    @functools.partial(
        pl.kernel,
        out_type=jax.ShapeDtypeStruct((NC, np_rows, d), F32),
        mesh=_mesh(),
        scratch_types=[
            pltpu.VMEM((C,), jnp.int32),
            pltpu.VMEM((C,), jnp.int32),
            pltpu.VMEM((C, d), F32),
            pltpu.VMEM((sb, d), F32),
            pltpu.VMEM_SHARED((np_rows, d), F32),
        ],
    )
    def k(feat_hbm, src_hbm, dst_hbm, out_hbm, sidx_v, didx_v, rows_v, stage_v, acc_sp):
        c = lax.axis_index("c")
        s = lax.axis_index("s")
        wid = s * NC + c

        @pl.loop(0, sb)
        def _(i):
            for j in range(d // LANES):
                stage_v[i, pl.ds(j * LANES, LANES)] = jnp.zeros((LANES,), F32)

        @pl.loop(0, nsb)
        def _(t):
            pltpu.sync_copy(stage_v, acc_sp.at[pl.ds(s * rpt + t * sb, sb)])

        plsc.subcore_barrier()

        @pl.loop(wid, nch, step=NC * NS)
        def _(ch):
            pltpu.sync_copy(src_hbm.at[ch], sidx_v)
            pltpu.sync_copy(feat_hbm.at[sidx_v], rows_v)
            pltpu.sync_copy(dst_hbm.at[ch], didx_v)
            pltpu.sync_copy(rows_v, acc_sp.at[didx_v], add=True)

        plsc.subcore_barrier()

        @pl.loop(0, nsb)
        def _(t):
            pltpu.sync_copy(acc_sp.at[pl.ds(s * rpt + t * sb, sb)], stage_v)
            pltpu.sync_copy(stage_v, out_hbm.at[c, pl.ds(s * rpt + t * sb, sb)])

    return k(feat, src2, dst2)


def _sc_gather_rows(table, idx2):
    np_rows, d = table.shape
    nch = idx2.shape[0]
    rpt = np_rows // NS
    sb = 64
    nsb = rpt // sb

    @functools.partial(
        pl.kernel,
        out_type=jax.ShapeDtypeStruct((nch * C, d), F32),
        mesh=_mesh(),
        scratch_types=[
            pltpu.VMEM((C,), jnp.int32),
            pltpu.VMEM((C, d), F32),
            pltpu.VMEM_SHARED((np_rows, d), F32),
        ],
    )
    def k(tab_hbm, idx_hbm, out_hbm, idx_v, rows_v, tab_sp):
        c = lax.axis_index("c")
        s = lax.axis_index("s")
        wid = s * NC + c

        @pl.loop(0, nsb)
        def _(t):
            pltpu.sync_copy(tab_hbm.at[pl.ds(s * rpt + t * sb, sb)], rows_v.at[pl.ds(0, sb)])
            pltpu.sync_copy(rows_v.at[pl.ds(0, sb)], tab_sp.at[pl.ds(s * rpt + t * sb, sb)])

        plsc.subcore_barrier()

        @pl.loop(wid, nch, step=NC * NS)
        def _(ch):
            pltpu.sync_copy(idx_hbm.at[ch], idx_v)
            pltpu.sync_copy(tab_sp.at[idx_v], rows_v)
            pltpu.sync_copy(rows_v, out_hbm.at[pl.ds(ch * C, C)])

    return k(table, idx2)


def _tc_norms_feat(degparts, h_pad):
    np_rows, d = h_pad.shape

    def body(dp_ref, h_ref, feat_ref, norms_ref):
        deg_s = dp_ref[0, :] + dp_ref[1, :]
        deg_d = dp_ref[2, :] + dp_ref[3, :]
        ns = lax.rsqrt(jnp.maximum(deg_s, 1.0))
        nd = lax.rsqrt(jnp.maximum(deg_d, 1.0))
        feat_ref[...] = h_ref[...] * ns[:, None]
        norms_ref[0, :] = ns
        norms_ref[1, :] = nd

    return pl.pallas_call(
        body,
        out_shape=(
            jax.ShapeDtypeStruct((np_rows, d), F32),
            jax.ShapeDtypeStruct((2, np_rows), F32),
        ),
    )(degparts, h_pad)


def _tc_layer(parts, norms, w, b2d, scale_src):
    np_rows, d = parts.shape[1], parts.shape[2]

    def body(p_ref, n_ref, w_ref, b_ref, o_ref):
        agg = (p_ref[0] + p_ref[1]) * n_ref[1, :][:, None]
        out = jnp.dot(agg, w_ref[...], preferred_element_type=F32,
                      precision=lax.Precision.HIGHEST)
        out = jnp.maximum(out + b_ref[...], 0.0)
        if scale_src:
            out = out * n_ref[0, :][:, None]
        o_ref[...] = out

    return pl.pallas_call(
        body,
        out_shape=jax.ShapeDtypeStruct((np_rows, w.shape[1]), F32),
    )(parts, norms, w, b2d)


def _tc_pairs(g, w3, b3_2d, p_count):
    d = g.shape[1]
    ncls = w3.shape[1]
    bp = 2000
    nblk = p_count // bp
    off2 = p_count // bp

    def body(h1_ref, h2_ref, w_ref, b_ref, o_ref):
        h1 = h1_ref[...]
        h2 = h2_ref[...]
        w = w_ref[...]
        out = jnp.dot(h1, w[0:d], preferred_element_type=F32,
                      precision=lax.Precision.HIGHEST)
        out += jnp.dot(h2, w[d:2 * d], preferred_element_type=F32,
                       precision=lax.Precision.HIGHEST)
        out += jnp.dot(jnp.abs(h1 - h2), w[2 * d:3 * d], preferred_element_type=F32,
                       precision=lax.Precision.HIGHEST)
        o_ref[...] = out + b_ref[...]

    return pl.pallas_call(
        body,
        grid=(nblk,),
        in_specs=[
            pl.BlockSpec((bp, d), lambda i: (i, 0)),
            pl.BlockSpec((bp, d), lambda i: (i + off2, 0)),
            pl.BlockSpec((3 * d, ncls), lambda i: (0, 0)),
            pl.BlockSpec((1, ncls), lambda i: (0, 0)),
        ],
        out_specs=pl.BlockSpec((bp, ncls), lambda i: (i, 0)),
        out_shape=jax.ShapeDtypeStruct((p_count, ncls), F32),
    )(g, g, w3, b3_2d)


def _pad_chunk_idx(idx, pad_value, chunk_mult=1):
    n = idx.shape[0]
    unit = C * chunk_mult
    n_pad = -(-n // unit) * unit
    if n_pad != n:
        idx = jnp.concatenate(
            [idx, jnp.full((n_pad - n,), pad_value, jnp.int32)])
    return idx.reshape(-1, C)


def kernel(h, edge_index, x1, x2, W1, b1, W2, b2, W3, b3):
    n, d = h.shape
    p_count = x1.shape[0]

    np_rows = -(-n // (NS * LANES * 8)) * (NS * LANES * 8)
    h_pad = jnp.pad(h, ((0, np_rows - n), (0, 0)))

    src2 = _pad_chunk_idx(edge_index[0], n)
    dst2 = _pad_chunk_idx(edge_index[1], n)
    x12 = _pad_chunk_idx(jnp.concatenate([x1, x2]), 0)

    degparts = _sc_degrees(jnp.stack([src2, dst2], axis=1), np_rows)
    feat1, norms = _tc_norms_feat(degparts, h_pad)
    parts1 = _sc_aggregate(feat1, src2, dst2)
    feat2 = _tc_layer(parts1, norms, W1, b1.reshape(1, -1), True)
    parts2 = _sc_aggregate(feat2, src2, dst2)
    h2arr = _tc_layer(parts2, norms, W2, b2.reshape(1, -1), False)
    g = _sc_gather_rows(h2arr, x12)
    return _tc_pairs(g, W3, b3.reshape(1, -1), p_count)


# speedup vs baseline: 1.9454x; 1.1043x over previous
import functools

import jax
import jax.numpy as jnp
from jax import lax
from jax.experimental import pallas as pl
from jax.experimental.pallas import tpu as pltpu
from jax.experimental.pallas import tpu_sc as plsc

NC = 2
NS = 16
LANES = 16
C = 128
F32 = jnp.float32


def _mesh():
    return plsc.VectorSubcoreMesh(
        core_axis_name="c", subcore_axis_name="s", num_cores=NC, num_subcores=NS
    )


def _sc_degrees(sd2, np_rows):
    nch = sd2.shape[0]
    npt = np_rows // NS

    @functools.partial(
        pl.kernel,
        out_type=jax.ShapeDtypeStruct((2 * NC, np_rows), F32),
        mesh=_mesh(),
        scratch_types=[
            pltpu.VMEM((2, C), jnp.int32),
            pltpu.VMEM((C,), F32),
            pltpu.VMEM((npt,), F32),
            pltpu.VMEM_SHARED((np_rows,), F32),
            pltpu.VMEM_SHARED((np_rows,), F32),
        ],
    )
    def k(sd_hbm, out_hbm, idx_v, ones_v, stage_v, degs_sp, degd_sp):
        c = lax.axis_index("c")
        s = lax.axis_index("s")
        wid = s * NC + c
        for j in range(C // LANES):
            ones_v[pl.ds(j * LANES, LANES)] = jnp.full((LANES,), 1.0, F32)
        for j in range(npt // LANES):
            stage_v[pl.ds(j * LANES, LANES)] = jnp.zeros((LANES,), F32)
        pltpu.sync_copy(stage_v, degs_sp.at[pl.ds(s * npt, npt)])
        pltpu.sync_copy(stage_v, degd_sp.at[pl.ds(s * npt, npt)])
        plsc.subcore_barrier()

        @pl.loop(wid, nch, step=NC * NS)
        def _(ch):
            pltpu.sync_copy(sd_hbm.at[ch], idx_v)
            pltpu.sync_copy(ones_v, degs_sp.at[idx_v.at[0]], add=True)
            pltpu.sync_copy(ones_v, degd_sp.at[idx_v.at[1]], add=True)

        plsc.subcore_barrier()
        pltpu.sync_copy(degs_sp.at[pl.ds(s * npt, npt)], stage_v)
        pltpu.sync_copy(stage_v, out_hbm.at[c, pl.ds(s * npt, npt)])
        pltpu.sync_copy(degd_sp.at[pl.ds(s * npt, npt)], stage_v)
        pltpu.sync_copy(stage_v, out_hbm.at[NC + c, pl.ds(s * npt, npt)])

    return k(sd2)


def _sc_aggregate(feat, sd2):
    np_rows, d = feat.shape
    nch = sd2.shape[0]
    rpt = np_rows // NS
    sb = 64
    nsb = rpt // sb

    @functools.partial(
        pl.kernel,
        out_type=jax.ShapeDtypeStruct((NC, np_rows, d), F32),
        mesh=_mesh(),
        scratch_types=[
            pltpu.VMEM((2, C), jnp.int32),
            pltpu.VMEM((C, d), F32),
            pltpu.VMEM((sb, d), F32),
            pltpu.VMEM_SHARED((np_rows, d), F32),
        ],
    )
    def k(feat_hbm, sd_hbm, out_hbm, idx_v, rows_v, stage_v, acc_sp):
        c = lax.axis_index("c")
        s = lax.axis_index("s")
        wid = s * NC + c

        @pl.loop(0, sb)
        def _(i):
            for j in range(d // LANES):
                stage_v[i, pl.ds(j * LANES, LANES)] = jnp.zeros((LANES,), F32)

        @pl.loop(0, nsb)
        def _(t):
            pltpu.sync_copy(stage_v, acc_sp.at[pl.ds(s * rpt + t * sb, sb)])

        plsc.subcore_barrier()

        @pl.loop(wid, nch, step=NC * NS)
        def _(ch):
            pltpu.sync_copy(sd_hbm.at[ch], idx_v)
            pltpu.sync_copy(feat_hbm.at[idx_v.at[0]], rows_v)
            pltpu.sync_copy(rows_v, acc_sp.at[idx_v.at[1]], add=True)

        plsc.subcore_barrier()

        @pl.loop(0, nsb)
        def _(t):
            pltpu.sync_copy(acc_sp.at[pl.ds(s * rpt + t * sb, sb)], stage_v)
            pltpu.sync_copy(stage_v, out_hbm.at[c, pl.ds(s * rpt + t * sb, sb)])

    return k(feat, sd2)


def _sc_gather_rows(table, idx2):
    np_rows, d = table.shape
    nch = idx2.shape[0]
    rpt = np_rows // NS
    sb = 64
    nsb = rpt // sb

    @functools.partial(
        pl.kernel,
        out_type=jax.ShapeDtypeStruct((nch * C, d), F32),
        mesh=_mesh(),
        scratch_types=[
            pltpu.VMEM((C,), jnp.int32),
            pltpu.VMEM((C, d), F32),
            pltpu.VMEM_SHARED((np_rows, d), F32),
        ],
    )
    def k(tab_hbm, idx_hbm, out_hbm, idx_v, rows_v, tab_sp):
        c = lax.axis_index("c")
        s = lax.axis_index("s")
        wid = s * NC + c

        @pl.loop(0, nsb)
        def _(t):
            pltpu.sync_copy(tab_hbm.at[pl.ds(s * rpt + t * sb, sb)], rows_v.at[pl.ds(0, sb)])
            pltpu.sync_copy(rows_v.at[pl.ds(0, sb)], tab_sp.at[pl.ds(s * rpt + t * sb, sb)])

        plsc.subcore_barrier()

        @pl.loop(wid, nch, step=NC * NS)
        def _(ch):
            pltpu.sync_copy(idx_hbm.at[ch], idx_v)
            pltpu.sync_copy(tab_sp.at[idx_v], rows_v)
            pltpu.sync_copy(rows_v, out_hbm.at[pl.ds(ch * C, C)])

    return k(table, idx2)


def _tc_norms_feat(degparts, h_pad):
    np_rows, d = h_pad.shape

    def body(dp_ref, h_ref, feat_ref, norms_ref):
        deg_s = dp_ref[0, :] + dp_ref[1, :]
        deg_d = dp_ref[2, :] + dp_ref[3, :]
        ns = lax.rsqrt(jnp.maximum(deg_s, 1.0))
        nd = lax.rsqrt(jnp.maximum(deg_d, 1.0))
        feat_ref[...] = h_ref[...] * ns[:, None]
        norms_ref[0, :] = ns
        norms_ref[1, :] = nd

    return pl.pallas_call(
        body,
        out_shape=(
            jax.ShapeDtypeStruct((np_rows, d), F32),
            jax.ShapeDtypeStruct((2, np_rows), F32),
        ),
    )(degparts, h_pad)


def _tc_layer(parts, norms, w, b2d, scale_src):
    np_rows, d = parts.shape[1], parts.shape[2]

    def body(p_ref, n_ref, w_ref, b_ref, o_ref):
        agg = (p_ref[0] + p_ref[1]) * n_ref[1, :][:, None]
        out = jnp.dot(agg, w_ref[...], preferred_element_type=F32,
                      precision=lax.Precision.HIGHEST)
        out = jnp.maximum(out + b_ref[...], 0.0)
        if scale_src:
            out = out * n_ref[0, :][:, None]
        o_ref[...] = out

    return pl.pallas_call(
        body,
        out_shape=jax.ShapeDtypeStruct((np_rows, w.shape[1]), F32),
    )(parts, norms, w, b2d)


def _tc_pairs(g, w3, b3_2d, p_count):
    d = g.shape[1]
    ncls = w3.shape[1]
    bp = 2000
    nblk = p_count // bp
    off2 = p_count // bp

    def body(h1_ref, h2_ref, w_ref, b_ref, o_ref):
        h1 = h1_ref[...]
        h2 = h2_ref[...]
        w = w_ref[...]
        out = jnp.dot(h1, w[0:d], preferred_element_type=F32,
                      precision=lax.Precision.HIGHEST)
        out += jnp.dot(h2, w[d:2 * d], preferred_element_type=F32,
                       precision=lax.Precision.HIGHEST)
        out += jnp.dot(jnp.abs(h1 - h2), w[2 * d:3 * d], preferred_element_type=F32,
                       precision=lax.Precision.HIGHEST)
        o_ref[...] = out + b_ref[...]

    return pl.pallas_call(
        body,
        grid=(nblk,),
        in_specs=[
            pl.BlockSpec((bp, d), lambda i: (i, 0)),
            pl.BlockSpec((bp, d), lambda i: (i + off2, 0)),
            pl.BlockSpec((3 * d, ncls), lambda i: (0, 0)),
            pl.BlockSpec((1, ncls), lambda i: (0, 0)),
        ],
        out_specs=pl.BlockSpec((bp, ncls), lambda i: (i, 0)),
        out_shape=jax.ShapeDtypeStruct((p_count, ncls), F32),
    )(g, g, w3, b3_2d)


def _pad_chunk_idx(idx, pad_value, chunk_mult=1):
    n = idx.shape[0]
    unit = C * chunk_mult
    n_pad = -(-n // unit) * unit
    if n_pad != n:
        idx = jnp.concatenate(
            [idx, jnp.full((n_pad - n,), pad_value, jnp.int32)])
    return idx.reshape(-1, C)


def kernel(h, edge_index, x1, x2, W1, b1, W2, b2, W3, b3):
    n, d = h.shape
    p_count = x1.shape[0]

    np_rows = -(-n // (NS * LANES * 8)) * (NS * LANES * 8)
    h_pad = jnp.pad(h, ((0, np_rows - n), (0, 0)))

    src2 = _pad_chunk_idx(edge_index[0], n)
    dst2 = _pad_chunk_idx(edge_index[1], n)
    x12 = _pad_chunk_idx(jnp.concatenate([x1, x2]), 0)

    sd2 = jnp.stack([src2, dst2], axis=1)
    degparts = _sc_degrees(sd2, np_rows)
    feat1, norms = _tc_norms_feat(degparts, h_pad)
    parts1 = _sc_aggregate(feat1, sd2)
    feat2 = _tc_layer(parts1, norms, W1, b1.reshape(1, -1), True)
    parts2 = _sc_aggregate(feat2, sd2)
    h2arr = _tc_layer(parts2, norms, W2, b2.reshape(1, -1), False)
    g = _sc_gather_rows(h2arr, x12)
    return _tc_pairs(g, W3, b3.reshape(1, -1), p_count)
